# baseline (speedup 1.0000x reference)
import functools

import jax
import jax.numpy as jnp
from jax import lax
from jax.experimental import pallas as pl
from jax.experimental.pallas import tpu as pltpu
from jax.experimental.pallas import tpu_sc as plsc

NUM_CORES = 2
NUM_SUBCORES = 16
LANES = 16
CHUNK = 128


def _sc_spmm(u_tiles, src, dst, w, n):
    e = w.shape[0]
    e_half = e // NUM_CORES
    chunks = e_half // CHUNK

    mesh = plsc.VectorSubcoreMesh(
        core_axis_name="c", subcore_axis_name="s",
        num_cores=NUM_CORES, num_subcores=NUM_SUBCORES)

    @functools.partial(
        pl.kernel,
        out_type=jax.ShapeDtypeStruct(
            (NUM_CORES, NUM_SUBCORES, n, LANES), jnp.float32),
        mesh=mesh,
        compiler_params=pltpu.CompilerParams(use_tc_tiling_on_sc=False),
        scratch_types=[
            pltpu.VMEM((CHUNK,), jnp.int32),
            pltpu.VMEM((CHUNK,), jnp.int32),
            pltpu.VMEM((CHUNK,), jnp.float32),
            pltpu.VMEM((CHUNK,), jnp.int32),
            pltpu.VMEM((CHUNK, LANES), jnp.float32),
            pltpu.VMEM((n, LANES), jnp.float32),
            pltpu.SemaphoreType.DMA,
        ],
    )
    def sc_kernel(u_hbm, src_hbm, dst_hbm, w_hbm, out_hbm,
                  dst_v, src_v, w_v, idx_v, rows_v, acc_v, sem):
        cid = lax.axis_index("c")
        sid = lax.axis_index("s")
        row0 = sid * n
        ebase = cid * e_half

        zero = jnp.zeros((LANES,), jnp.float32)

        def zero_row(r, _):
            acc_v[r] = zero
            return 0

        lax.fori_loop(0, n, zero_row, 0)

        def do_chunk(c, _):
            off = ebase + c * CHUNK
            pltpu.sync_copy(dst_hbm.at[pl.ds(off, CHUNK)], dst_v)
            pltpu.sync_copy(src_hbm.at[pl.ds(off, CHUNK)], src_v)
            pltpu.sync_copy(w_hbm.at[pl.ds(off, CHUNK)], w_v)

            def mkidx(g, _):
                sl = pl.ds(g * LANES, LANES)
                idx_v[sl] = dst_v[sl] + row0
                return 0

            lax.fori_loop(0, CHUNK // LANES, mkidx, 0)
            pltpu.async_copy(u_hbm.at[idx_v], rows_v, sem).wait()

            def acc_group(g, _):
                base = g * LANES
                wg = w_v[pl.ds(base, LANES)]
                sg = src_v[pl.ds(base, LANES)]
                for j in range(LANES):
                    v = rows_v[base + j]
                    plsc.addupdate(acc_v.at[sg[j]], v * wg[j])
                return 0

            lax.fori_loop(0, CHUNK // LANES, acc_group, 0)
            return 0

        lax.fori_loop(0, chunks, do_chunk, 0)
        pltpu.sync_copy(acc_v, out_hbm.at[cid, sid])

    return sc_kernel(u_tiles, src, dst, w)


TC_BLK = 512


def _tc_finish_body(acc_ref, w_ref, b_ref, out_ref, ssq_ref):
    i = pl.program_id(0)

    @pl.when(i == 0)
    def _():
        ssq_ref[0] = 0.0

    a = acc_ref[0] + acc_ref[1]
    half = NUM_SUBCORES // 2
    left = jnp.concatenate([a[t] for t in range(half)], axis=1)
    right = jnp.concatenate([a[t] for t in range(half, NUM_SUBCORES)], axis=1)
    s = jnp.dot(left, w_ref[...], preferred_element_type=jnp.float32) + b_ref[...]
    t = jnp.dot(right, w_ref[...], preferred_element_type=jnp.float32) + b_ref[...]
    diff = jnp.maximum(s, 0.0) - jnp.maximum(t, 0.0)
    ssq_ref[0] += jnp.sum(diff * diff)

    @pl.when(i == pl.num_programs(0) - 1)
    def _():
        out_ref[...] = jnp.exp(-jnp.sqrt(ssq_ref[0])).reshape(1, 1)


def kernel(x, edge_index, edge_weight, feat_a, W, b):
    n, d_in = x.shape
    d_hid = W.shape[1]
    u = jnp.concatenate([x, feat_a], axis=1)
    u_tiles = u.reshape(n, NUM_SUBCORES, LANES).transpose(1, 0, 2)
    u_tiles = u_tiles.reshape(NUM_SUBCORES * n, LANES)
    acc = _sc_spmm(u_tiles, edge_index[0], edge_index[1], edge_weight, n)
    out = pl.pallas_call(
        _tc_finish_body,
        grid=(n // TC_BLK,),
        in_specs=[
            pl.BlockSpec((NUM_CORES, NUM_SUBCORES, TC_BLK, LANES),
                         lambda i: (0, 0, i, 0)),
            pl.BlockSpec((d_in, d_hid), lambda i: (0, 0)),
            pl.BlockSpec((1, d_hid), lambda i: (0, 0)),
        ],
        out_specs=pl.BlockSpec((1, 1), lambda i: (0, 0)),
        out_shape=jax.ShapeDtypeStruct((1, 1), jnp.float32),
        scratch_shapes=[pltpu.SMEM((1,), jnp.float32)],
    )(acc, W, b.reshape(1, -1))
    return out[0, 0]
